# baseline (speedup 1.0000x reference)
import math
import functools

import jax
import jax.numpy as jnp
from jax import lax
from jax.experimental import pallas as pl
from jax.experimental.pallas import tpu as pltpu
from jax.experimental.pallas import tpu_sc as plsc

_DIM = 1024
_NEG_LOG_FREQ_OVER_DIM = -math.log(10000.0) / _DIM
_SUB = 256
_NBASE = 16


def _pe_table_block(out_ref, sr_ref, cr_ref, ca_ref, cb_ref, *, s_blk):
    i = pl.program_id(0)

    @pl.when(i == 0)
    def _init_scratch():
        lane = jax.lax.broadcasted_iota(jnp.int32, (16, _DIM), 1)
        even = (lane % 2) == 0
        inv_freq = jnp.exp((lane - (lane % 2)).astype(jnp.float32)
                           * _NEG_LOG_FREQ_OVER_DIM)
        j = jax.lax.broadcasted_iota(jnp.int32, (16, _DIM), 0)
        jf = j.astype(jnp.float32) * inv_freq
        s_lo = jnp.sin(jf)
        c_lo = jnp.cos(jf)
        qf = jf * 16.0
        s_hi = jnp.sin(qf)
        c_hi = jnp.cos(qf)
        for q in range(16):
            sq = s_hi[q:q + 1, :]
            cq = c_hi[q:q + 1, :]
            sl = pl.ds(q * 16, 16)
            sr_ref[sl, :] = sq * c_lo + cq * s_lo
            cr_ref[sl, :] = cq * c_lo - sq * s_lo
        bf = jf * 256.0
        s_b = jnp.sin(bf)
        c_b = jnp.cos(bf)
        ca_ref[...] = jnp.where(even, c_b, -s_b)
        cb_ref[...] = jnp.where(even, s_b, c_b)

    sr = sr_ref[...]
    cr = cr_ref[...]
    n_sub = s_blk // _SUB
    for a in range(n_sub):
        k = (i * n_sub + a) % _NBASE
        ca = ca_ref[pl.ds(k, 1), :]
        cb = cb_ref[pl.ds(k, 1), :]
        out_ref[pl.ds(a * _SUB, _SUB), :] = sr * ca + cr * cb


def _make_pe_table(max_len):
    s_blk = 2048
    return pl.pallas_call(
        functools.partial(_pe_table_block, s_blk=s_blk),
        grid=(max_len // s_blk,),
        out_specs=pl.BlockSpec((s_blk, _DIM), lambda i: (i, 0)),
        out_shape=jax.ShapeDtypeStruct((max_len, _DIM), jnp.float32),
        scratch_shapes=[
            pltpu.VMEM((_SUB, _DIM), jnp.float32),
            pltpu.VMEM((_SUB, _DIM), jnp.float32),
            pltpu.VMEM((_NBASE, _DIM), jnp.float32),
            pltpu.VMEM((_NBASE, _DIM), jnp.float32),
        ],
    )()


_CHUNK = 16384


def _make_sc_add(total, pe_total):
    info = plsc.get_sparse_core_info()
    nw = info.num_cores * info.num_subcores
    per_w = total // nw
    n_chunks = per_w // _CHUNK
    mesh = plsc.VectorSubcoreMesh(core_axis_name="c", subcore_axis_name="s")

    @functools.partial(
        pl.kernel, mesh=mesh,
        out_type=jax.ShapeDtypeStruct((total,), jnp.float32),
        scratch_types=[
            pltpu.VMEM((_CHUNK,), jnp.float32),
            pltpu.VMEM((_CHUNK,), jnp.float32),
        ],
    )
    def sc_add(emb_hbm, pe_hbm, out_hbm, emb_v, pe_v):
        wid = lax.axis_index("s") * info.num_cores + lax.axis_index("c")
        base = wid * per_w

        def chunk_body(c, carry):
            off = base + c * _CHUNK
            pe_off = lax.rem(off, pe_total)
            pltpu.sync_copy(emb_hbm.at[pl.ds(off, _CHUNK)], emb_v)
            pltpu.sync_copy(pe_hbm.at[pl.ds(pe_off, _CHUNK)], pe_v)

            def add_body(i, carry2):
                s = pl.ds(i * 16, 16)
                emb_v[s] = emb_v[s] + pe_v[s]
                return carry2

            lax.fori_loop(0, _CHUNK // 16, add_body, 0)
            pltpu.sync_copy(emb_v, out_hbm.at[pl.ds(off, _CHUNK)])
            return carry

        lax.fori_loop(0, n_chunks, chunk_body, 0)

    return sc_add


@jax.jit
def kernel(position_ids, embeddings):
    batch, max_len, dim = embeddings.shape
    pe = _make_pe_table(max_len)
    total = batch * max_len * dim
    flat = embeddings.reshape(total)
    out = _make_sc_add(total, max_len * dim)(flat, pe.reshape(-1))
    return out.reshape(batch, max_len, dim)


# speedup vs baseline: 9.6712x; 9.6712x over previous
import math
import functools

import jax
import jax.numpy as jnp
from jax.experimental import pallas as pl
from jax.experimental.pallas import tpu as pltpu

_DIM = 1024
_NEG_LOG_FREQ_OVER_DIM = -math.log(10000.0) / _DIM
_SUB = 256
_NBASE = 16


def _pe_add_block(emb_ref, out_ref, sr_ref, cr_ref, ca_ref, cb_ref,
                  *, s_blk, max_len):
    i = pl.program_id(0)

    @pl.when(i == 0)
    def _init_scratch():
        lane = jax.lax.broadcasted_iota(jnp.int32, (16, _DIM), 1)
        even = (lane % 2) == 0
        inv_freq = jnp.exp((lane - (lane % 2)).astype(jnp.float32)
                           * _NEG_LOG_FREQ_OVER_DIM)
        j = jax.lax.broadcasted_iota(jnp.int32, (16, _DIM), 0)
        jf = j.astype(jnp.float32) * inv_freq
        s_lo = jnp.sin(jf)
        c_lo = jnp.cos(jf)
        qf = jf * 16.0
        s_hi = jnp.sin(qf)
        c_hi = jnp.cos(qf)
        for q in range(16):
            sq = s_hi[q:q + 1, :]
            cq = c_hi[q:q + 1, :]
            sl = pl.ds(q * 16, 16)
            sr_ref[sl, :] = sq * c_lo + cq * s_lo
            cr_ref[sl, :] = cq * c_lo - sq * s_lo
        s_b, c_b = s_hi, c_hi
        for _ in range(4):
            s_b, c_b = 2.0 * s_b * c_b, c_b * c_b - s_b * s_b
        ca_ref[...] = jnp.where(even, c_b, -s_b)
        cb_ref[...] = jnp.where(even, s_b, c_b)

    sr = sr_ref[...]
    cr = cr_ref[...]
    n_sub = s_blk // _SUB
    for a in range(n_sub):
        k = (i * n_sub + a) % _NBASE
        ca = ca_ref[pl.ds(k, 1), :]
        cb = cb_ref[pl.ds(k, 1), :]
        sl = pl.ds(a * _SUB, _SUB)
        out_ref[sl, :] = (emb_ref[sl, :] + sr * ca) + cr * cb


@jax.jit
def kernel(position_ids, embeddings):
    batch, max_len, dim = embeddings.shape
    s_blk = 2048
    flat = embeddings.reshape(batch * max_len, dim)
    grid = (flat.shape[0] // s_blk,)
    out = pl.pallas_call(
        functools.partial(_pe_add_block, s_blk=s_blk, max_len=max_len),
        grid=grid,
        in_specs=[pl.BlockSpec((s_blk, dim), lambda i: (i, 0))],
        out_specs=pl.BlockSpec((s_blk, dim), lambda i: (i, 0)),
        out_shape=jax.ShapeDtypeStruct(flat.shape, flat.dtype),
        scratch_shapes=[
            pltpu.VMEM((_SUB, _DIM), jnp.float32),
            pltpu.VMEM((_SUB, _DIM), jnp.float32),
            pltpu.VMEM((_NBASE, _DIM), jnp.float32),
            pltpu.VMEM((_NBASE, _DIM), jnp.float32),
        ],
    )(flat)
    return out.reshape(batch, max_len, dim)
